# Initial kernel scaffold; baseline (speedup 1.0000x reference)
import jax
import jax.numpy as jnp
from jax.experimental import pallas as pl


def kernel(x, edge_index, edge_type, params):
    raise NotImplementedError("write your pallas kernel here")


# speedup vs baseline: 7.1695x; 7.1695x over previous
import functools

import jax
import jax.numpy as jnp
from jax import lax
from jax.experimental import pallas as pl
from jax.experimental.pallas import tpu as pltpu
from jax.experimental.pallas import tpu_sc as plsc

_EPS = 1e-15

_NC = 2
_NS = 16
_ZB = 80


def _row_norm(u):
    return jnp.maximum(jnp.sqrt(jnp.sum(u * u, axis=1, keepdims=True)), _EPS)


def _expmap0(u):
    n = _row_norm(u)
    return jnp.tanh(n) * u / n


def _logmap0(u):
    n = _row_norm(u)
    c = jnp.clip(n, -1.0 + 1e-7, 1.0 - 1e-7)
    return 0.5 * (jnp.log1p(c) - jnp.log1p(-c)) * u / n


def _gidx_body(n_nodes, src_ref, et_ref, gidx_ref):
    gidx_ref[...] = et_ref[...] * n_nodes + src_ref[...]


def _node_body(h_ref, ws_ref, bs_ref, wr_ref, br_ref, xs_ref, z_ref):
    num_rel = wr_ref.shape[0]
    h = h_ref[...]
    t = _logmap0(_expmap0(h))

    def hyp(w, b):
        u = lax.dot_general(t, w, (((1,), (1,)), ((), ())),
                            preferred_element_type=jnp.float32,
                            precision=lax.Precision.HIGHEST) + b
        return _expmap0(u)

    xs_ref[...] = hyp(ws_ref[...], bs_ref[...])
    for r in range(num_rel):
        z_ref[r] = hyp(wr_ref[r], br_ref[r])


def _combine_body(xs_ref, p_ref, h_ref):
    x = xs_ref[...]
    y = p_ref[0] + p_ref[1]
    x2 = jnp.sum(x * x, axis=1, keepdims=True)
    y2 = jnp.sum(y * y, axis=1, keepdims=True)
    xy = jnp.sum(x * y, axis=1, keepdims=True)
    num = (1.0 + 2.0 * xy + y2) * x + (1.0 - x2) * y
    den = jnp.maximum(1.0 + 2.0 * xy + x2 * y2, _EPS)
    h_ref[...] = _logmap0(num / den)


def _make_edge_kernel(n_nodes, d, n_edges):
    nw = _NC * _NS
    e_pt = n_edges // nw
    ch = 80
    n_chunks = e_pt // ch
    cz = _ZB
    mesh = plsc.VectorSubcoreMesh(core_axis_name="c", subcore_axis_name="s")

    @functools.partial(
        pl.kernel,
        out_type=jax.ShapeDtypeStruct((_NC, n_nodes, d), jnp.float32),
        mesh=mesh,
        scratch_types=[
            pltpu.VMEM((ch,), jnp.int32),
            pltpu.VMEM((ch,), jnp.int32),
            pltpu.VMEM((ch, d), jnp.float32),
            pltpu.VMEM_SHARED((n_nodes, d), jnp.float32),
            pltpu.SemaphoreType.DMA,
        ],
    )
    def edge_kernel(table_hbm, gidx_hbm, dst_hbm, zrow_hbm, out_hbm,
                    gidx_v, dst_v, rows_v, acc_sh, sem):
        cid = lax.axis_index("c")
        sid = lax.axis_index("s")
        wid = sid * _NC + cid

        @pl.loop(sid * cz, n_nodes, step=_NS * cz)
        def _(r):
            pltpu.sync_copy(zrow_hbm, acc_sh.at[pl.ds(r, cz)])

        plsc.subcore_barrier()

        base = wid * e_pt

        @pl.loop(0, n_chunks)
        def _(j):
            off = base + j * ch
            pltpu.sync_copy(gidx_hbm.at[pl.ds(off, ch)], gidx_v)
            pltpu.sync_copy(dst_hbm.at[pl.ds(off, ch)], dst_v)
            pltpu.async_copy(table_hbm.at[gidx_v], rows_v, sem).wait()
            pltpu.sync_copy(rows_v, acc_sh.at[dst_v], add=True)

        plsc.subcore_barrier()

        @pl.loop(sid * cz, n_nodes, step=_NS * cz)
        def _(r):
            pltpu.sync_copy(acc_sh.at[pl.ds(r, cz)],
                            out_hbm.at[cid, pl.ds(r, cz)])

    return edge_kernel


def kernel(x, edge_index, edge_type, params):
    n, d = x.shape
    e = edge_index.shape[1]
    num_rel = len(params[0]["w_rel"])
    src = edge_index[0]
    dst = edge_index[1]

    ecols = 128
    gidx2d = pl.pallas_call(
        functools.partial(_gidx_body, n),
        out_shape=jax.ShapeDtypeStruct((e // ecols, ecols), jnp.int32),
    )(src.reshape(e // ecols, ecols), edge_type.reshape(e // ecols, ecols))
    gidx = gidx2d.reshape(e)

    zrow = jnp.zeros((_ZB, d), jnp.float32)
    edge_call = _make_edge_kernel(n, d, e)

    bn = 1000
    grid = n // bn
    node_call = pl.pallas_call(
        _node_body,
        grid=(grid,),
        in_specs=[
            pl.BlockSpec((bn, d), lambda i: (i, 0)),
            pl.BlockSpec((d, d), lambda i: (0, 0)),
            pl.BlockSpec((1, d), lambda i: (0, 0)),
            pl.BlockSpec((num_rel, d, d), lambda i: (0, 0, 0)),
            pl.BlockSpec((num_rel, 1, d), lambda i: (0, 0, 0)),
        ],
        out_specs=[
            pl.BlockSpec((bn, d), lambda i: (i, 0)),
            pl.BlockSpec((num_rel, bn, d), lambda i: (0, i, 0)),
        ],
        out_shape=[
            jax.ShapeDtypeStruct((n, d), jnp.float32),
            jax.ShapeDtypeStruct((num_rel, n, d), jnp.float32),
        ],
    )
    combine_call = pl.pallas_call(
        _combine_body,
        grid=(grid,),
        in_specs=[
            pl.BlockSpec((bn, d), lambda i: (i, 0)),
            pl.BlockSpec((_NC, bn, d), lambda i: (0, i, 0)),
        ],
        out_specs=pl.BlockSpec((bn, d), lambda i: (i, 0)),
        out_shape=jax.ShapeDtypeStruct((n, d), jnp.float32),
    )

    h = x
    for p in params:
        ws = p["w_self"]
        bs = p["b_self"].reshape(1, d)
        wr = jnp.stack(p["w_rel"])
        br = jnp.stack(p["b_rel"]).reshape(num_rel, 1, d)
        x_self, z = node_call(h, ws, bs, wr, br)
        partials = edge_call(z.reshape(num_rel * n, d), gidx, dst, zrow)
        h = combine_call(x_self, partials)
    return h


# speedup vs baseline: 11.7965x; 1.6454x over previous
import functools

import jax
import jax.numpy as jnp
from jax import lax
from jax.experimental import pallas as pl
from jax.experimental.pallas import tpu as pltpu
from jax.experimental.pallas import tpu_sc as plsc

_EPS = 1e-15

_NC = 2
_NS = 16
_ZB = 80
_CH = 50
_NBLK = 10


def _row_norm(u):
    return jnp.maximum(jnp.sqrt(jnp.sum(u * u, axis=1, keepdims=True)), _EPS)


def _expmap0(u):
    n = _row_norm(u)
    return jnp.tanh(n) * u / n


def _logmap0(u):
    n = _row_norm(u)
    c = jnp.clip(n, -1.0 + 1e-7, 1.0 - 1e-7)
    return 0.5 * (jnp.log1p(c) - jnp.log1p(-c)) * u / n


def _gidx_body(n_nodes, src_ref, et_ref, gidx_ref):
    gidx_ref[...] = et_ref[...] * n_nodes + src_ref[...]


def _node_body(h_ref, ws_ref, bs_ref, wr_ref, br_ref, xs_ref, z_ref):
    num_rel = wr_ref.shape[0]
    h = h_ref[...]
    t = _logmap0(_expmap0(h))

    def hyp(w, b):
        u = lax.dot_general(t, w, (((1,), (1,)), ((), ())),
                            preferred_element_type=jnp.float32,
                            precision=lax.Precision.HIGHEST) + b
        return _expmap0(u)

    xs_ref[...] = hyp(ws_ref[...], bs_ref[...])
    for r in range(num_rel):
        z_ref[r] = hyp(wr_ref[r], br_ref[r])


def _combine_body(xs_ref, p_ref, h_ref):
    x = xs_ref[...]
    y = p_ref[0] + p_ref[1]
    x2 = jnp.sum(x * x, axis=1, keepdims=True)
    y2 = jnp.sum(y * y, axis=1, keepdims=True)
    xy = jnp.sum(x * y, axis=1, keepdims=True)
    num = (1.0 + 2.0 * xy + y2) * x + (1.0 - x2) * y
    den = jnp.maximum(1.0 + 2.0 * xy + x2 * y2, _EPS)
    h_ref[...] = _logmap0(num / den)


def _make_edge_kernel(n_nodes, d, n_edges):
    nw = _NC * _NS
    e_pt = n_edges // nw
    ch = _CH
    nblk = _NBLK
    blk = e_pt // ch // nblk
    cz = _ZB
    mesh = plsc.VectorSubcoreMesh(core_axis_name="c", subcore_axis_name="s")

    @functools.partial(
        pl.kernel,
        out_type=jax.ShapeDtypeStruct((_NC, n_nodes, d), jnp.float32),
        mesh=mesh,
        scratch_types=[
            pltpu.VMEM((2, blk, ch), jnp.int32),
            pltpu.VMEM((2, blk, ch), jnp.int32),
            pltpu.VMEM((2, ch, d), jnp.float32),
            pltpu.VMEM_SHARED((n_nodes, d), jnp.float32),
            [pltpu.SemaphoreType.DMA] * 2,
            [pltpu.SemaphoreType.DMA] * 2,
        ],
    )
    def edge_kernel(table_hbm, gidx_hbm, dst_hbm, zrow_hbm, out_hbm,
                    gidx_v, dst_v, rows_v, acc_sh, isem, gsem):
        cid = lax.axis_index("c")
        sid = lax.axis_index("s")
        wid = sid * _NC + cid

        def load_idx(g, bp, sync):
            if sync:
                pltpu.sync_copy(gidx_hbm.at[wid, g], gidx_v.at[bp])
                pltpu.sync_copy(dst_hbm.at[wid, g], dst_v.at[bp])
            else:
                pltpu.async_copy(gidx_hbm.at[wid, g], gidx_v.at[bp], isem[bp])
                pltpu.async_copy(dst_hbm.at[wid, g], dst_v.at[bp], isem[bp])

        def wait_idx(g, bp):
            pltpu.make_async_copy(gidx_hbm.at[wid, g], gidx_v.at[bp],
                                  isem[bp]).wait()
            pltpu.make_async_copy(dst_hbm.at[wid, g], dst_v.at[bp],
                                  isem[bp]).wait()

        def gather(bp, jj, rb):
            return pltpu.async_copy(
                table_hbm.at[gidx_v.at[bp, jj]], rows_v.at[rb], gsem[rb])

        def wait_gather(bp, jj, rb):
            pltpu.make_async_copy(
                table_hbm.at[gidx_v.at[bp, jj]], rows_v.at[rb],
                gsem[rb]).wait()

        def scatter_add(bp, jj, rb):
            pltpu.sync_copy(rows_v.at[rb], acc_sh.at[dst_v.at[bp, jj]],
                            add=True)

        def block_body(g, bp, bq, last):
            @pl.loop(0, blk - 2, step=2)
            def _(jj):
                for u in range(2):
                    wait_gather(bp, jj + u, u)
                    scatter_add(bp, jj + u, u)
                    gather(bp, jj + u + 2, u)
            if not last:
                wait_idx(g + 1, bq)
            for u in range(2):
                wait_gather(bp, blk - 2 + u, u)
                scatter_add(bp, blk - 2 + u, u)
                if not last:
                    gather(bq, u, u)

        load_idx(0, 0, sync=False)

        @pl.loop(sid * cz, n_nodes, step=_NS * cz)
        def _(r):
            pltpu.sync_copy(zrow_hbm, acc_sh.at[pl.ds(r, cz)])

        wait_idx(0, 0)
        plsc.subcore_barrier()

        load_idx(1, 1, sync=False)
        gather(0, 0, 0)
        gather(0, 1, 1)

        @pl.loop(0, nblk - 2, step=2)
        def _(g):
            block_body(g, 0, 1, last=False)
            load_idx(g + 2, 0, sync=False)
            block_body(g + 1, 1, 0, last=False)
            load_idx(g + 3, 1, sync=False)

        block_body(nblk - 2, 0, 1, last=False)
        block_body(nblk - 1, 1, 0, last=True)

        plsc.subcore_barrier()

        @pl.loop(sid * cz, n_nodes, step=_NS * cz)
        def _(r):
            pltpu.sync_copy(acc_sh.at[pl.ds(r, cz)],
                            out_hbm.at[cid, pl.ds(r, cz)])

    return edge_kernel


def kernel(x, edge_index, edge_type, params):
    n, d = x.shape
    e = edge_index.shape[1]
    num_rel = len(params[0]["w_rel"])
    src = edge_index[0]
    dst = edge_index[1]

    ecols = 128
    gidx2d = pl.pallas_call(
        functools.partial(_gidx_body, n),
        out_shape=jax.ShapeDtypeStruct((e // ecols, ecols), jnp.int32),
    )(src.reshape(e // ecols, ecols), edge_type.reshape(e // ecols, ecols))
    nw = _NC * _NS
    blk = (e // nw) // _CH // _NBLK
    gidx3 = gidx2d.reshape(nw, _NBLK, blk, _CH)
    dst3 = dst.reshape(nw, _NBLK, blk, _CH)

    zrow = jnp.zeros((_ZB, d), jnp.float32)
    edge_call = _make_edge_kernel(n, d, e)

    bn = 1000
    grid = n // bn
    node_call = pl.pallas_call(
        _node_body,
        grid=(grid,),
        in_specs=[
            pl.BlockSpec((bn, d), lambda i: (i, 0)),
            pl.BlockSpec((d, d), lambda i: (0, 0)),
            pl.BlockSpec((1, d), lambda i: (0, 0)),
            pl.BlockSpec((num_rel, d, d), lambda i: (0, 0, 0)),
            pl.BlockSpec((num_rel, 1, d), lambda i: (0, 0, 0)),
        ],
        out_specs=[
            pl.BlockSpec((bn, d), lambda i: (i, 0)),
            pl.BlockSpec((num_rel, bn, d), lambda i: (0, i, 0)),
        ],
        out_shape=[
            jax.ShapeDtypeStruct((n, d), jnp.float32),
            jax.ShapeDtypeStruct((num_rel, n, d), jnp.float32),
        ],
    )
    combine_call = pl.pallas_call(
        _combine_body,
        grid=(grid,),
        in_specs=[
            pl.BlockSpec((bn, d), lambda i: (i, 0)),
            pl.BlockSpec((_NC, bn, d), lambda i: (0, i, 0)),
        ],
        out_specs=pl.BlockSpec((bn, d), lambda i: (i, 0)),
        out_shape=jax.ShapeDtypeStruct((n, d), jnp.float32),
    )

    h = x
    for p in params:
        ws = p["w_self"]
        bs = p["b_self"].reshape(1, d)
        wr = jnp.stack(p["w_rel"])
        br = jnp.stack(p["b_rel"]).reshape(num_rel, 1, d)
        x_self, z = node_call(h, ws, bs, wr, br)
        partials = edge_call(z.reshape(num_rel * n, d), gidx3, dst3, zrow)
        h = combine_call(x_self, partials)
    return h


# speedup vs baseline: 13.4198x; 1.1376x over previous
import functools

import jax
import jax.numpy as jnp
from jax import lax
from jax.experimental import pallas as pl
from jax.experimental.pallas import tpu as pltpu
from jax.experimental.pallas import tpu_sc as plsc

_EPS = 1e-15

_NC = 2
_NS = 16
_ZB = 400
_CH = 50
_NBLK = 10


def _row_norm(u):
    return jnp.maximum(jnp.sqrt(jnp.sum(u * u, axis=1, keepdims=True)), _EPS)


def _expmap0(u):
    n = _row_norm(u)
    return jnp.tanh(n) * u / n


def _logmap0(u):
    n = _row_norm(u)
    c = jnp.clip(n, -1.0 + 1e-7, 1.0 - 1e-7)
    return 0.5 * (jnp.log1p(c) - jnp.log1p(-c)) * u / n


def _gidx_body(n_nodes, src_ref, et_ref, gidx_ref):
    gidx_ref[...] = et_ref[...] * n_nodes + src_ref[...]


def _combine_math(x, y):
    x2 = jnp.sum(x * x, axis=1, keepdims=True)
    y2 = jnp.sum(y * y, axis=1, keepdims=True)
    xy = jnp.sum(x * y, axis=1, keepdims=True)
    num = (1.0 + 2.0 * xy + y2) * x + (1.0 - x2) * y
    den = jnp.maximum(1.0 + 2.0 * xy + x2 * y2, _EPS)
    return _logmap0(num / den)


def _emit_node(h, ws_ref, bs_ref, wr_ref, br_ref, xs_ref, z_ref):
    num_rel = wr_ref.shape[0]
    t = _logmap0(_expmap0(h))

    def hyp(w, b):
        u = lax.dot_general(t, w, (((1,), (1,)), ((), ())),
                            preferred_element_type=jnp.float32,
                            precision=lax.Precision.HIGHEST) + b
        return _expmap0(u)

    xs_ref[...] = hyp(ws_ref[...], bs_ref[...])
    for r in range(num_rel):
        z_ref[r] = hyp(wr_ref[r], br_ref[r])


def _node_body(h_ref, ws_ref, bs_ref, wr_ref, br_ref, xs_ref, z_ref):
    _emit_node(h_ref[...], ws_ref, bs_ref, wr_ref, br_ref, xs_ref, z_ref)


def _node0_body(n_nodes, h_ref, ws_ref, bs_ref, wr_ref, br_ref,
                src_ref, et_ref, xs_ref, z_ref, gidx_ref):
    gidx_ref[...] = et_ref[...] * n_nodes + src_ref[...]
    _emit_node(h_ref[...], ws_ref, bs_ref, wr_ref, br_ref, xs_ref, z_ref)


def _mid_body(xs_ref, p_ref, ws_ref, bs_ref, wr_ref, br_ref,
              xs2_ref, z_ref):
    h = _combine_math(xs_ref[...], p_ref[0] + p_ref[1])
    _emit_node(h, ws_ref, bs_ref, wr_ref, br_ref, xs2_ref, z_ref)


def _combine_body(xs_ref, p_ref, h_ref):
    h_ref[...] = _combine_math(xs_ref[...], p_ref[0] + p_ref[1])


def _make_edge_kernel(n_nodes, d, n_edges):
    nw = _NC * _NS
    e_pt = n_edges // nw
    ch = _CH
    nblk = _NBLK
    blk = e_pt // ch // nblk
    cz = _ZB
    mesh = plsc.VectorSubcoreMesh(core_axis_name="c", subcore_axis_name="s")

    @functools.partial(
        pl.kernel,
        out_type=jax.ShapeDtypeStruct((_NC, n_nodes, d), jnp.float32),
        mesh=mesh,
        scratch_types=[
            pltpu.VMEM((2, blk, ch), jnp.int32),
            pltpu.VMEM((2, blk, ch), jnp.int32),
            pltpu.VMEM((2, ch, d), jnp.float32),
            pltpu.VMEM_SHARED((n_nodes, d), jnp.float32),
            [pltpu.SemaphoreType.DMA] * 2,
            [pltpu.SemaphoreType.DMA] * 2,
        ],
    )
    def edge_kernel(table_hbm, gidx_hbm, dst_hbm, zrow_hbm, out_hbm,
                    gidx_v, dst_v, rows_v, acc_sh, isem, gsem):
        cid = lax.axis_index("c")
        sid = lax.axis_index("s")
        wid = sid * _NC + cid

        def load_idx(g, bp, sync):
            if sync:
                pltpu.sync_copy(gidx_hbm.at[wid, g], gidx_v.at[bp])
                pltpu.sync_copy(dst_hbm.at[wid, g], dst_v.at[bp])
            else:
                pltpu.async_copy(gidx_hbm.at[wid, g], gidx_v.at[bp], isem[bp])
                pltpu.async_copy(dst_hbm.at[wid, g], dst_v.at[bp], isem[bp])

        def wait_idx(g, bp):
            pltpu.make_async_copy(gidx_hbm.at[wid, g], gidx_v.at[bp],
                                  isem[bp]).wait()
            pltpu.make_async_copy(dst_hbm.at[wid, g], dst_v.at[bp],
                                  isem[bp]).wait()

        def gather(bp, jj, rb):
            return pltpu.async_copy(
                table_hbm.at[gidx_v.at[bp, jj]], rows_v.at[rb], gsem[rb])

        def wait_gather(bp, jj, rb):
            pltpu.make_async_copy(
                table_hbm.at[gidx_v.at[bp, jj]], rows_v.at[rb],
                gsem[rb]).wait()

        def scatter_add(bp, jj, rb):
            pltpu.sync_copy(rows_v.at[rb], acc_sh.at[dst_v.at[bp, jj]],
                            add=True)

        def block_body(g, bp, bq, last):
            @pl.loop(0, blk - 2, step=2)
            def _(jj):
                for u in range(2):
                    wait_gather(bp, jj + u, u)
                    scatter_add(bp, jj + u, u)
                    gather(bp, jj + u + 2, u)
            if not last:
                wait_idx(g + 1, bq)
            for u in range(2):
                wait_gather(bp, blk - 2 + u, u)
                scatter_add(bp, blk - 2 + u, u)
                if not last:
                    gather(bq, u, u)

        load_idx(0, 0, sync=False)

        @pl.loop(sid * cz, n_nodes, step=_NS * cz)
        def _(r):
            pltpu.sync_copy(zrow_hbm, acc_sh.at[pl.ds(r, cz)])

        wait_idx(0, 0)
        plsc.subcore_barrier()

        load_idx(1, 1, sync=False)
        gather(0, 0, 0)
        gather(0, 1, 1)

        @pl.loop(0, nblk - 2, step=2)
        def _(g):
            block_body(g, 0, 1, last=False)
            load_idx(g + 2, 0, sync=False)
            block_body(g + 1, 1, 0, last=False)
            load_idx(g + 3, 1, sync=False)

        block_body(nblk - 2, 0, 1, last=False)
        block_body(nblk - 1, 1, 0, last=True)

        plsc.subcore_barrier()

        @pl.loop(sid * cz, n_nodes, step=_NS * cz)
        def _(r):
            pltpu.sync_copy(acc_sh.at[pl.ds(r, cz)],
                            out_hbm.at[cid, pl.ds(r, cz)])

    return edge_kernel


def kernel(x, edge_index, edge_type, params):
    n, d = x.shape
    e = edge_index.shape[1]
    num_rel = len(params[0]["w_rel"])
    src = edge_index[0]
    dst = edge_index[1]

    nw = _NC * _NS
    ecols = 128
    blk = (e // nw) // _CH // _NBLK
    dst3 = dst.reshape(nw, _NBLK, blk, _CH)

    zrow = jnp.zeros((_ZB, d), jnp.float32)
    edge_call = _make_edge_kernel(n, d, e)

    bn = 2000
    grid = n // bn
    row_spec = pl.BlockSpec((bn, d), lambda i: (i, 0))
    par_spec = pl.BlockSpec((_NC, bn, d), lambda i: (0, i, 0))
    w_specs = [
        pl.BlockSpec((d, d), lambda i: (0, 0)),
        pl.BlockSpec((1, d), lambda i: (0, 0)),
        pl.BlockSpec((num_rel, d, d), lambda i: (0, 0, 0)),
        pl.BlockSpec((num_rel, 1, d), lambda i: (0, 0, 0)),
    ]
    node_outs = dict(
        out_specs=[
            row_spec,
            pl.BlockSpec((num_rel, bn, d), lambda i: (0, i, 0)),
        ],
        out_shape=[
            jax.ShapeDtypeStruct((n, d), jnp.float32),
            jax.ShapeDtypeStruct((num_rel, n, d), jnp.float32),
        ],
    )
    erows = e // ecols
    eb = erows // grid
    e_spec = pl.BlockSpec((1, eb, ecols), lambda i: (i, 0, 0))
    node0_call = pl.pallas_call(
        functools.partial(_node0_body, n), grid=(grid,),
        in_specs=[row_spec] + w_specs + [e_spec, e_spec],
        out_specs=node_outs["out_specs"] + [e_spec],
        out_shape=node_outs["out_shape"]
        + [jax.ShapeDtypeStruct((grid, eb, ecols), jnp.int32)],
    )
    mid_call = pl.pallas_call(
        _mid_body, grid=(grid,),
        in_specs=[row_spec, par_spec] + w_specs, **node_outs)
    combine_call = pl.pallas_call(
        _combine_body, grid=(grid,),
        in_specs=[row_spec, par_spec],
        out_specs=row_spec,
        out_shape=jax.ShapeDtypeStruct((n, d), jnp.float32),
    )

    def wts(p):
        return (p["w_self"], p["b_self"].reshape(1, d),
                jnp.stack(p["w_rel"]),
                jnp.stack(p["b_rel"]).reshape(num_rel, 1, d))

    x_self, z, gidx2d = node0_call(
        x, *wts(params[0]), src.reshape(grid, eb, ecols),
        edge_type.reshape(grid, eb, ecols))
    gidx3 = gidx2d.reshape(nw, _NBLK, blk, _CH)
    for p in params[1:]:
        partials = edge_call(z.reshape(num_rel * n, d), gidx3, dst3, zrow)
        x_self, z = mid_call(x_self, partials, *wts(p))
    partials = edge_call(z.reshape(num_rel * n, d), gidx3, dst3, zrow)
    return combine_call(x_self, partials)
